# baseline (speedup 1.0000x reference)
import jax
import jax.numpy as jnp
from jax import lax
from jax.experimental import pallas as pl
from jax.experimental.pallas import tpu as pltpu
from jax.experimental.pallas import tpu_sc as plsc

B = 2
CIN = 32
COUT = 16
D1 = 4
N_IN = 16384
N_OUT = 16384
F = 15
H = 32768
RLOC = H + 1
HP = 32832
TLOC = B * HP
RPT = TLOC // 16
NC = 2
NS = 16
CH = 128
PC = 512

_mesh = plsc.VectorSubcoreMesh(core_axis_name="c", subcore_axis_name="s",
                               num_cores=NC, num_subcores=NS)


def _splat_body(feats_t, bary, offs, table_out,
                table_sh, counts_sh, fbuf, wbuf, ibuf, idxb, pay, cnt1, sem):
    c = lax.axis_index("c")
    s = lax.axis_index("s")
    rbase = s * RPT
    _ZERO16 = jnp.zeros((16,), jnp.float32)

    def _z(i, _):
        pay[i, :] = _ZERO16
        return 0
    lax.fori_loop(0, PC, _z, 0)

    def _zw(j, _):
        cnt1[pl.ds(j * 16, 16)] = _ZERO16
        return 0
    lax.fori_loop(0, PC // 16, _zw, 0)

    def _zcopy(i, _):
        pltpu.sync_copy(pay, table_sh.at[pl.ds(rbase + i * PC, PC), :])
        pltpu.sync_copy(cnt1, counts_sh.at[pl.ds(rbase + i * PC, PC)])
        return 0
    lax.fori_loop(0, 8, _zcopy, 0)
    pltpu.sync_copy(pay.at[pl.ds(0, 8), :],
                    table_sh.at[pl.ds(rbase + 8 * PC, 8), :])
    pltpu.sync_copy(cnt1.at[pl.ds(0, 8)],
                    counts_sh.at[pl.ds(rbase + 8 * PC, 8)])
    plsc.subcore_barrier()

    def _chunk(b, k):
        n0 = s * (N_IN // NS) + k * PC
        loads = [pltpu.async_copy(
            feats_t.at[b, pl.ds(n0, PC), pl.ds(c * 16, 16)], fbuf, sem)]
        for d in range(D1):
            f0 = (b * D1 + d) * N_IN + n0
            loads.append(pltpu.async_copy(offs.at[pl.ds(f0, PC)],
                                          ibuf.at[d], sem))
            loads.append(pltpu.async_copy(bary.at[pl.ds(f0, PC)],
                                          wbuf.at[d], sem))
        for ld in loads:
            ld.wait()
        for d in range(D1):
            def _idx(j, _, d=d):
                q = j // 8
                g = ibuf[d, pl.ds(j * 16, 16)] + 1
                idxb[q, pl.ds((j - q * 8) * 16, 16)] = jnp.where(
                    g >= RLOC, g + (HP - RLOC), g)
                return 0
            lax.fori_loop(0, PC // 16, _idx, 0)

            def _pay(p, _, d=d):
                ws = plsc.load_gather(
                    wbuf, [jnp.zeros((16,), jnp.int32) + d,
                           jnp.zeros((16,), jnp.int32) + p])
                pay[p, :] = ws * fbuf[p, :]
                return 0
            lax.fori_loop(0, PC, _pay, 0)

            scs = []
            for q in range(PC // CH):
                scs.append(pltpu.async_copy(
                    pay.at[pl.ds(q * CH, CH), :],
                    table_sh.at[idxb.at[q]], sem, add=True))
                scs.append(pltpu.async_copy(
                    wbuf.at[d, pl.ds(q * CH, CH)],
                    counts_sh.at[idxb.at[q]], sem, add=True))
            for sc in scs:
                sc.wait()

    for b in range(B):
        lax.fori_loop(0, (N_IN // NS) // PC,
                      lambda k, _, b=b: (_chunk(b, k), 0)[1], 0)
    plsc.subcore_barrier()

    def _norm_flush(r0, nr):
        l0 = pltpu.async_copy(counts_sh.at[pl.ds(r0, nr)],
                              cnt1.at[pl.ds(0, nr)], sem)
        l1 = pltpu.async_copy(table_sh.at[pl.ds(r0, nr), :],
                              pay.at[pl.ds(0, nr), :], sem)
        l0.wait()
        l1.wait()

        def _n(i, _):
            cnt = plsc.load_gather(cnt1, [jnp.zeros((16,), jnp.int32) + i])
            pay[i, :] = pay[i, :] * (1.0 / (cnt + 1e-5))
            return 0
        lax.fori_loop(0, nr, _n, 0)
        pltpu.sync_copy(pay.at[pl.ds(0, nr), :],
                        table_out.at[c, pl.ds(r0, nr), :])

    lax.fori_loop(0, 8, lambda i, _: (_norm_flush(rbase + i * PC, PC), 0)[1], 0)
    _norm_flush(rbase + 8 * PC, 8)


_SC_PARAMS = pltpu.CompilerParams(needs_layout_passes=False,
                                  use_tc_tiling_on_sc=False)

_splat = pl.kernel(
    _splat_body,
    out_type=jax.ShapeDtypeStruct((NC, TLOC, 16), jnp.float32),
    mesh=_mesh,
    compiler_params=_SC_PARAMS,
    scratch_types=[
        pltpu.VMEM_SHARED((TLOC, 16), jnp.float32),
        pltpu.VMEM_SHARED((TLOC,), jnp.float32),
        pltpu.VMEM((PC, 16), jnp.float32),
        pltpu.VMEM((D1, PC), jnp.float32),
        pltpu.VMEM((D1, PC), jnp.int32),
        pltpu.VMEM((PC // CH, CH), jnp.int32),
        pltpu.VMEM((PC, 16), jnp.float32),
        pltpu.VMEM((PC,), jnp.float32),
        pltpu.SemaphoreType.DMA,
    ],
)


def _mm_body(xlo_ref, xhi_ref, wlo_ref, whi_ref, bc_ref, o_ref):
    xlo = xlo_ref[0]
    xhi = xhi_ref[0]
    for f in range(F):
        o_ref[0, f] = (jnp.dot(xlo, wlo_ref[f],
                               preferred_element_type=jnp.float32)
                       + jnp.dot(xhi, whi_ref[f],
                                 preferred_element_type=jnp.float32)
                       + bc_ref[:])


def _blur_conv(xlo, xhi, wlo, whi, bc8):
    blk8 = 456
    return pl.pallas_call(
        _mm_body,
        grid=(B, (HP // 8) // blk8),
        in_specs=[
            pl.BlockSpec((1, blk8, 128), lambda b, i: (b, i, 0)),
            pl.BlockSpec((1, blk8, 128), lambda b, i: (b, i, 0)),
            pl.BlockSpec((F, 128, 128), lambda b, i: (0, 0, 0)),
            pl.BlockSpec((F, 128, 128), lambda b, i: (0, 0, 0)),
            pl.BlockSpec((1, 128), lambda b, i: (0, 0)),
        ],
        out_specs=pl.BlockSpec((1, F, blk8, 128), lambda b, i: (b, 0, i, 0)),
        out_shape=jax.ShapeDtypeStruct((B, F, HP // 8, 128), jnp.float32),
    )(xlo, xhi, wlo, whi, bc8)


def _blur_slice_body(y2, bn, ooffs, obary, fb2, out_t,
                     ibuf, idxg, wbuf, gbuf, acc, sem):
    c = lax.axis_index("c")
    s = lax.axis_index("s")

    def _hchunk(k, _):
        h0 = s * (H // NS) + k * CH
        loads = [pltpu.async_copy(bn.at[pl.ds((c * F + f) * H + h0, CH)],
                                  ibuf.at[f], sem) for f in range(F)]
        for ld in loads:
            ld.wait()
        for f in range(F):
            base = (c * F + f) * HP + 1

            def _gi(j, _, f=f, base=base):
                idxg[f, pl.ds(j * 16, 16)] = ibuf[f, pl.ds(j * 16, 16)] + base
                return 0
            lax.fori_loop(0, CH // 16, _gi, 0)
        gets = [pltpu.async_copy(y2.at[idxg.at[f]], gbuf.at[f], sem)
                for f in range(F)]
        for g in gets:
            g.wait()

        def _acc(i, _):
            v = gbuf[0, i, :]
            for f in range(1, F):
                v = v + gbuf[f, i, :]
            acc[i, :] = v
            return 0
        lax.fori_loop(0, CH, _acc, 0)
        pltpu.sync_copy(acc, fb2.at[pl.ds(c * H + h0, CH), :])
        return 0
    lax.fori_loop(0, (H // NS) // CH, _hchunk, 0)
    plsc.subcore_barrier()

    def _pchunk(m, _):
        n0 = s * (N_OUT // NS) + m * CH
        loads = []
        for d in range(D1):
            f0 = (c * D1 + d) * N_OUT + n0
            loads.append(pltpu.async_copy(ooffs.at[pl.ds(f0, CH)],
                                          ibuf.at[d], sem))
            loads.append(pltpu.async_copy(obary.at[pl.ds(f0, CH)],
                                          wbuf.at[d], sem))
        for ld in loads:
            ld.wait()
        for d in range(D1):
            def _gi(j, _, d=d):
                idxg[d, pl.ds(j * 16, 16)] = ibuf[d, pl.ds(j * 16, 16)] + c * H
                return 0
            lax.fori_loop(0, CH // 16, _gi, 0)
        gets = [pltpu.async_copy(fb2.at[idxg.at[d]], gbuf.at[d], sem)
                for d in range(D1)]
        for g in gets:
            g.wait()

        def _w(p, _):
            pz = jnp.zeros((16,), jnp.int32) + p
            v = plsc.load_gather(wbuf, [jnp.zeros((16,), jnp.int32), pz]) \
                * gbuf[0, p, :]
            for d in range(1, D1):
                v = v + plsc.load_gather(
                    wbuf, [jnp.zeros((16,), jnp.int32) + d, pz]) * gbuf[d, p, :]
            acc[p, :] = v
            return 0
        lax.fori_loop(0, CH, _w, 0)
        pltpu.sync_copy(acc, out_t.at[c, pl.ds(n0, CH), :])
        return 0
    lax.fori_loop(0, (N_OUT // NS) // CH, _pchunk, 0)


_blur_slice = pl.kernel(
    _blur_slice_body,
    out_type=(jax.ShapeDtypeStruct((B * H, COUT), jnp.float32),
              jax.ShapeDtypeStruct((B, N_OUT, COUT), jnp.float32)),
    mesh=_mesh,
    compiler_params=_SC_PARAMS,
    scratch_types=[
        pltpu.VMEM((F, CH), jnp.int32),
        pltpu.VMEM((F, CH), jnp.int32),
        pltpu.VMEM((D1, CH), jnp.float32),
        pltpu.VMEM((F, CH, COUT), jnp.float32),
        pltpu.VMEM((CH, COUT), jnp.float32),
        pltpu.SemaphoreType.DMA,
    ],
)


@jax.jit
def kernel(features, in_barycentric, in_lattice_offset, blur_neighbors,
           out_barycentric, out_lattice_offset, W, b_conv, bias):
    feats_t = jnp.transpose(features, (0, 2, 1))
    wt = jnp.transpose(W, (2, 1, 0))
    eye8 = jnp.eye(8, dtype=W.dtype)
    wlo = (jnp.einsum('ab,fco->facbo', eye8, wt[:, :16, :])
           .reshape(F, 128, 128))
    whi = (jnp.einsum('ab,fco->facbo', eye8, wt[:, 16:, :])
           .reshape(F, 128, 128))
    bc8 = jnp.tile(b_conv, 8).reshape(1, 128)
    tbl = _splat(feats_t, in_barycentric.reshape(-1),
                 in_lattice_offset.reshape(-1))
    xlo = tbl[0].reshape(B, HP // 8, 128)
    xhi = tbl[1].reshape(B, HP // 8, 128)
    y = _blur_conv(xlo, xhi, wlo, whi, bc8)
    y2 = y.reshape(B * F * HP, COUT)
    _, out_t = _blur_slice(y2, blur_neighbors.reshape(-1),
                           out_lattice_offset.reshape(-1),
                           out_barycentric.reshape(-1))
    return jnp.transpose(out_t, (0, 2, 1)) + bias[None, :, None]


# speedup vs baseline: 1.2293x; 1.2293x over previous
import jax
import jax.numpy as jnp
from jax import lax
from jax.experimental import pallas as pl
from jax.experimental.pallas import tpu as pltpu
from jax.experimental.pallas import tpu_sc as plsc

B = 2
CIN = 32
COUT = 16
D1 = 4
N_IN = 16384
N_OUT = 16384
F = 15
H = 32768
RLOC = H + 1
TLOC = 32896
HP = 33280
NC = 2
NS = 16
CH = 128
PC = 512

_mesh = plsc.VectorSubcoreMesh(core_axis_name="c", subcore_axis_name="s",
                               num_cores=NC, num_subcores=NS)


def _splat_body(feats_t, bary, offs, table_out,
                table_sh, counts_sh, fbuf, wbuf, ibuf, idxb, pay, cnt1, sem):
    c = lax.axis_index("c")
    s = lax.axis_index("s")
    rbase = s * 2056
    _ZERO16 = jnp.zeros((16,), jnp.float32)

    def _z(i, _):
        pay[i, pl.ds(0, 16)] = _ZERO16
        pay[i, pl.ds(16, 16)] = _ZERO16
        return 0
    lax.fori_loop(0, PC, _z, 0)

    def _zw(j, _):
        cnt1[pl.ds(j * 16, 16)] = _ZERO16
        return 0
    lax.fori_loop(0, PC // 16, _zw, 0)

    def _zcopy(i, _):
        pltpu.sync_copy(pay, table_sh.at[pl.ds(rbase + i * PC, PC), :])
        pltpu.sync_copy(cnt1, counts_sh.at[pl.ds(rbase + i * PC, PC)])
        return 0
    lax.fori_loop(0, 4, _zcopy, 0)
    pltpu.sync_copy(pay.at[pl.ds(0, 8), :],
                    table_sh.at[pl.ds(rbase + 4 * PC, 8), :])
    pltpu.sync_copy(cnt1.at[pl.ds(0, 8)],
                    counts_sh.at[pl.ds(rbase + 4 * PC, 8)])
    plsc.subcore_barrier()

    shift = 1 - c * RLOC

    def _chunk(b, k):
        n0 = s * (N_IN // NS) + k * PC
        loads = [pltpu.async_copy(feats_t.at[b, pl.ds(n0, PC), :], fbuf, sem)]
        for d in range(D1):
            f0 = (b * D1 + d) * N_IN + n0
            loads.append(pltpu.async_copy(offs.at[pl.ds(f0, PC)],
                                          ibuf.at[d], sem))
            loads.append(pltpu.async_copy(bary.at[pl.ds(f0, PC)],
                                          wbuf.at[d], sem))
        for ld in loads:
            ld.wait()
        for d in range(D1):
            def _idx(j, _, d=d):
                q = j // 8
                g = ibuf[d, pl.ds(j * 16, 16)] + shift
                ok = (g >= 0) & (g < RLOC)
                idxb[q, pl.ds((j - q * 8) * 16, 16)] = jnp.where(ok, g, RLOC)
                return 0
            lax.fori_loop(0, PC // 16, _idx, 0)

            def _pay(p, _, d=d):
                ws = plsc.load_gather(
                    wbuf, [jnp.zeros((16,), jnp.int32) + d,
                           jnp.zeros((16,), jnp.int32) + p])
                pay[p, pl.ds(0, 16)] = ws * fbuf[p, pl.ds(0, 16)]
                pay[p, pl.ds(16, 16)] = ws * fbuf[p, pl.ds(16, 16)]
                return 0
            lax.fori_loop(0, PC, _pay, 0)

            scs = []
            for q in range(PC // CH):
                scs.append(pltpu.async_copy(
                    pay.at[pl.ds(q * CH, CH), :],
                    table_sh.at[idxb.at[q]], sem, add=True))
                scs.append(pltpu.async_copy(
                    wbuf.at[d, pl.ds(q * CH, CH)],
                    counts_sh.at[idxb.at[q]], sem, add=True))
            for sc in scs:
                sc.wait()

    for b in range(B):
        lax.fori_loop(0, (N_IN // NS) // PC,
                      lambda k, _, b=b: (_chunk(b, k), 0)[1], 0)
    plsc.subcore_barrier()

    def _norm_flush(r0, nr):
        l0 = pltpu.async_copy(counts_sh.at[pl.ds(r0, nr)],
                              cnt1.at[pl.ds(0, nr)], sem)
        l1 = pltpu.async_copy(table_sh.at[pl.ds(r0, nr), :],
                              pay.at[pl.ds(0, nr), :], sem)
        l0.wait()
        l1.wait()

        def _n(i, _):
            cnt = plsc.load_gather(cnt1, [jnp.zeros((16,), jnp.int32) + i])
            nrm = 1.0 / (cnt + 1e-5)
            pay[i, pl.ds(0, 16)] = pay[i, pl.ds(0, 16)] * nrm
            pay[i, pl.ds(16, 16)] = pay[i, pl.ds(16, 16)] * nrm
            return 0
        lax.fori_loop(0, nr, _n, 0)
        pltpu.sync_copy(pay.at[pl.ds(0, nr), :],
                        table_out.at[c, pl.ds(r0, nr), :])

    lax.fori_loop(0, 4, lambda i, _: (_norm_flush(rbase + i * PC, PC), 0)[1], 0)
    _norm_flush(rbase + 4 * PC, 8)


_SC_PARAMS = pltpu.CompilerParams(needs_layout_passes=False,
                                  use_tc_tiling_on_sc=False)

_splat = pl.kernel(
    _splat_body,
    out_type=jax.ShapeDtypeStruct((B, HP, CIN), jnp.float32),
    mesh=_mesh,
    compiler_params=_SC_PARAMS,
    scratch_types=[
        pltpu.VMEM_SHARED((TLOC, CIN), jnp.float32),
        pltpu.VMEM_SHARED((TLOC,), jnp.float32),
        pltpu.VMEM((PC, CIN), jnp.float32),
        pltpu.VMEM((D1, PC), jnp.float32),
        pltpu.VMEM((D1, PC), jnp.int32),
        pltpu.VMEM((PC // CH, CH), jnp.int32),
        pltpu.VMEM((PC, CIN), jnp.float32),
        pltpu.VMEM((PC,), jnp.float32),
        pltpu.SemaphoreType.DMA,
    ],
)


def _mm_body(x_ref, w_ref, bc_ref, o_ref):
    x8 = x_ref[0]
    for f in range(F):
        o_ref[0, f] = (jnp.dot(x8, w_ref[f],
                               preferred_element_type=jnp.float32)
                       + bc_ref[:])


def _blur_conv(table, w3, bc8):
    blk = 2560
    return pl.pallas_call(
        _mm_body,
        grid=(B, HP // blk),
        in_specs=[
            pl.BlockSpec((1, blk // 8, 8 * CIN), lambda b, i: (b, i, 0)),
            pl.BlockSpec((F, 8 * CIN, 128), lambda b, i: (0, 0, 0)),
            pl.BlockSpec((1, 128), lambda b, i: (0, 0)),
        ],
        out_specs=pl.BlockSpec((1, F, blk // 8, 128), lambda b, i: (b, 0, i, 0)),
        out_shape=jax.ShapeDtypeStruct((B, F, HP // 8, 128), jnp.float32),
    )(table.reshape(B, HP // 8, 8 * CIN), w3, bc8)


def _blur_slice_body(y2, bn, ooffs, obary, fb2, out_t,
                     ibuf, idxg, wbuf, gbuf, acc, sem):
    c = lax.axis_index("c")
    s = lax.axis_index("s")

    def _hchunk(k, _):
        h0 = s * (H // NS) + k * CH
        loads = [pltpu.async_copy(bn.at[pl.ds((c * F + f) * H + h0, CH)],
                                  ibuf.at[f], sem) for f in range(F)]
        for ld in loads:
            ld.wait()
        for f in range(F):
            base = (c * F + f) * HP + 1

            def _gi(j, _, f=f, base=base):
                idxg[f, pl.ds(j * 16, 16)] = ibuf[f, pl.ds(j * 16, 16)] + base
                return 0
            lax.fori_loop(0, CH // 16, _gi, 0)
        gets = [pltpu.async_copy(y2.at[idxg.at[f]], gbuf.at[f], sem)
                for f in range(F)]
        for g in gets:
            g.wait()

        def _acc(i, _):
            v = gbuf[0, i, :]
            for f in range(1, F):
                v = v + gbuf[f, i, :]
            acc[i, :] = v
            return 0
        lax.fori_loop(0, CH, _acc, 0)
        pltpu.sync_copy(acc, fb2.at[pl.ds(c * H + h0, CH), :])
        return 0
    lax.fori_loop(0, (H // NS) // CH, _hchunk, 0)
    plsc.subcore_barrier()

    def _pchunk(m, _):
        n0 = s * (N_OUT // NS) + m * CH
        loads = []
        for d in range(D1):
            f0 = (c * D1 + d) * N_OUT + n0
            loads.append(pltpu.async_copy(ooffs.at[pl.ds(f0, CH)],
                                          ibuf.at[d], sem))
            loads.append(pltpu.async_copy(obary.at[pl.ds(f0, CH)],
                                          wbuf.at[d], sem))
        for ld in loads:
            ld.wait()
        for d in range(D1):
            def _gi(j, _, d=d):
                idxg[d, pl.ds(j * 16, 16)] = ibuf[d, pl.ds(j * 16, 16)] + c * H
                return 0
            lax.fori_loop(0, CH // 16, _gi, 0)
        gets = [pltpu.async_copy(fb2.at[idxg.at[d]], gbuf.at[d], sem)
                for d in range(D1)]
        for g in gets:
            g.wait()

        def _w(p, _):
            pz = jnp.zeros((16,), jnp.int32) + p
            v = plsc.load_gather(wbuf, [jnp.zeros((16,), jnp.int32), pz]) \
                * gbuf[0, p, :]
            for d in range(1, D1):
                v = v + plsc.load_gather(
                    wbuf, [jnp.zeros((16,), jnp.int32) + d, pz]) * gbuf[d, p, :]
            acc[p, :] = v
            return 0
        lax.fori_loop(0, CH, _w, 0)
        pltpu.sync_copy(acc, out_t.at[c, pl.ds(n0, CH), :])
        return 0
    lax.fori_loop(0, (N_OUT // NS) // CH, _pchunk, 0)


_blur_slice = pl.kernel(
    _blur_slice_body,
    out_type=(jax.ShapeDtypeStruct((B * H, COUT), jnp.float32),
              jax.ShapeDtypeStruct((B, N_OUT, COUT), jnp.float32)),
    mesh=_mesh,
    compiler_params=_SC_PARAMS,
    scratch_types=[
        pltpu.VMEM((F, CH), jnp.int32),
        pltpu.VMEM((F, CH), jnp.int32),
        pltpu.VMEM((D1, CH), jnp.float32),
        pltpu.VMEM((F, CH, COUT), jnp.float32),
        pltpu.VMEM((CH, COUT), jnp.float32),
        pltpu.SemaphoreType.DMA,
    ],
)


@jax.jit
def kernel(features, in_barycentric, in_lattice_offset, blur_neighbors,
           out_barycentric, out_lattice_offset, W, b_conv, bias):
    feats_t = jnp.transpose(features, (0, 2, 1))
    wt = jnp.transpose(W, (2, 1, 0))
    w3 = (jnp.einsum('ab,fco->facbo', jnp.eye(8, dtype=W.dtype), wt)
          .reshape(F, 8 * CIN, 8 * COUT))
    bc8 = jnp.tile(b_conv, 8).reshape(1, 128)
    table = _splat(feats_t, in_barycentric.reshape(-1),
                   in_lattice_offset.reshape(-1))
    y = _blur_conv(table, w3, bc8)
    y2 = y.reshape(B * F * HP, COUT)
    _, out_t = _blur_slice(y2, blur_neighbors.reshape(-1),
                           out_lattice_offset.reshape(-1),
                           out_barycentric.reshape(-1))
    return jnp.transpose(out_t, (0, 2, 1)) + bias[None, :, None]
